# baseline (speedup 1.0000x reference)
import functools

import jax
import jax.numpy as jnp
from jax import lax
from jax.experimental import pallas as pl
from jax.experimental.pallas import tpu as pltpu
from jax.experimental.pallas import tpu_sc as plsc

_NUM_CORES = 2
_NUM_SUBCORES = 16
_NW = _NUM_CORES * _NUM_SUBCORES
_NBUF = 4
_SPLITS = (0, 56, 104, 152, 200)
_PADS = 256
_L = 16


@functools.lru_cache(maxsize=None)
def _build(n_b, n_s, dim):
    rows_per_w = n_b // _NW
    widths = tuple(_SPLITS[i + 1] - _SPLITS[i] for i in range(len(_SPLITS) - 1))
    steps_per_row = len(widths)
    bufw = max(widths)
    mesh = plsc.VectorSubcoreMesh(core_axis_name="c", subcore_axis_name="s")

    @functools.partial(
        pl.kernel,
        mesh=mesh,
        out_type=jax.ShapeDtypeStruct((n_b, n_s, dim), jnp.float32),
        scratch_types=(
            [
                pltpu.VMEM((rows_per_w * _PADS,), jnp.int32),
                pltpu.VMEM((rows_per_w * _PADS,), jnp.int32),
            ]
            + [pltpu.VMEM((bufw, 2 * dim), jnp.float32) for _ in range(_NBUF)]
            + [pltpu.VMEM((bufw, dim), jnp.float32) for _ in range(_NBUF)]
            + [pltpu.SemaphoreType.DMA for _ in range(2 * _NBUF)]
        ),
        compiler_params=pltpu.CompilerParams(
            use_tc_tiling_on_sc=True, skip_device_barrier=True
        ),
    )
    def run(idx2_hbm, off_hbm, table_hbm, out_hbm, idx2_v, off_v, *bufs_and_sems):
        bufs = bufs_and_sems[:_NBUF]
        obufs = bufs_and_sems[_NBUF : 2 * _NBUF]
        gsems = bufs_and_sems[2 * _NBUF : 3 * _NBUF]
        osems = bufs_and_sems[3 * _NBUF :]
        wid = lax.axis_index("s") * _NUM_CORES + lax.axis_index("c")
        row0 = wid * rows_per_w
        pltpu.sync_copy(
            idx2_hbm.at[pl.ds(row0 * _PADS, rows_per_w * _PADS)], idx2_v
        )
        pltpu.sync_copy(off_hbm.at[pl.ds(row0 * _PADS, rows_per_w * _PADS)], off_v)

        def gather(r, h, b):
            c0, w = _SPLITS[h], widths[h]
            return pltpu.make_async_copy(
                table_hbm.at[idx2_v.at[pl.ds(r * _PADS + c0, w)]],
                bufs[b].at[pl.ds(0, w)],
                gsems[b],
            )

        def store(r, h, b):
            c0, w = _SPLITS[h], widths[h]
            return pltpu.make_async_copy(
                obufs[b].at[pl.ds(0, w)],
                out_hbm.at[row0 + r, pl.ds(c0, w)],
                osems[b],
            )

        def repack(r, h, b):
            c0, w = _SPLITS[h], widths[h]
            buf, obuf = bufs[b], obufs[b]
            for k0 in range(0, w, _L):
                nl = min(_L, w - k0)
                off_vec = off_v[pl.ds(r * _PADS + c0 + k0, _L)]
                for t in range(nl):
                    off = off_vec[t]
                    k = k0 + t
                    for c4 in range(dim // _L):
                        obuf[k, pl.ds(c4 * _L, _L)] = buf[
                            k, pl.ds(off + c4 * _L, _L)
                        ]

        for b in range(_NBUF):
            gather(b // steps_per_row, b % steps_per_row, b).start()

        rows_per_group = _NBUF // steps_per_row if _NBUF >= steps_per_row else 1
        assert _NBUF == steps_per_row

        def loop_body(r, carry):
            for b in range(_NBUF):
                gather(r, b, b).wait()
                repack(r, b, b)
                store(r, b, b).start()
            for b in range(_NBUF):
                store(r, b, b).wait()
                nr = r + 1

                @pl.when(nr < rows_per_w)
                def _():
                    gather(nr, b, b).start()

            return carry

        lax.fori_loop(0, rows_per_w, loop_body, 0)

    return run


def kernel(token_ids, weight):
    n_b, n_s = token_ids.shape
    dim = weight.shape[1]
    t = token_ids.astype(jnp.int32)
    pad = ((0, 0), (0, _PADS - n_s))
    idx2 = jnp.pad(t >> 1, pad).reshape(-1)
    off = jnp.pad((t & 1) * dim, pad).reshape(-1)
    pair_table = weight.reshape(weight.shape[0] // 2, 2 * dim)
    return _build(n_b, n_s, dim)(idx2, off, pair_table)


# speedup vs baseline: 1.0428x; 1.0428x over previous
import functools

import jax
import jax.numpy as jnp
from jax import lax
from jax.experimental import pallas as pl
from jax.experimental.pallas import tpu as pltpu
from jax.experimental.pallas import tpu_sc as plsc

_NUM_CORES = 2
_NUM_SUBCORES = 16
_NW = _NUM_CORES * _NUM_SUBCORES
_NBUF = 4
_SPLITS = (0, 104, 200)


@functools.lru_cache(maxsize=None)
def _build(n_b, n_s, dim):
    rows_per_w = n_b // _NW
    mesh = plsc.VectorSubcoreMesh(core_axis_name="c", subcore_axis_name="s")
    widths = tuple(
        _SPLITS[i + 1] - _SPLITS[i] for i in range(len(_SPLITS) - 1)
    )
    steps_per_row = len(widths)

    @functools.partial(
        pl.kernel,
        mesh=mesh,
        out_type=jax.ShapeDtypeStruct((n_b, n_s, dim), jnp.float32),
        scratch_types=(
            [pltpu.VMEM((rows_per_w, n_s), jnp.int32)]
            + [pltpu.VMEM((max(widths), dim), jnp.float32) for _ in range(_NBUF)]
            + [pltpu.SemaphoreType.DMA for _ in range(2 * _NBUF)]
        ),
        compiler_params=pltpu.CompilerParams(
            use_tc_tiling_on_sc=False, skip_device_barrier=True
        ),
    )
    def run(idx_hbm, table_hbm, out_hbm, idx_v, *bufs_and_sems):
        bufs = bufs_and_sems[:_NBUF]
        gsems = bufs_and_sems[_NBUF : 2 * _NBUF]
        osems = bufs_and_sems[2 * _NBUF :]
        wid = lax.axis_index("s") * _NUM_CORES + lax.axis_index("c")
        row0 = wid * rows_per_w
        pltpu.sync_copy(idx_hbm.at[pl.ds(row0, rows_per_w)], idx_v)

        rows_per_group = _NBUF // steps_per_row

        def gather(r, h, b):
            c0, w = _SPLITS[h], widths[h]
            return pltpu.make_async_copy(
                table_hbm.at[idx_v.at[r, pl.ds(c0, w)]],
                bufs[b].at[pl.ds(0, w)],
                gsems[b],
            )

        def store(r, h, b):
            c0, w = _SPLITS[h], widths[h]
            return pltpu.make_async_copy(
                bufs[b].at[pl.ds(0, w)],
                out_hbm.at[row0 + r, pl.ds(c0, w)],
                osems[b],
            )

        for b in range(_NBUF):
            gather(b // steps_per_row, b % steps_per_row, b).start()

        def loop_body(g, carry):
            r0 = g * rows_per_group
            for b in range(_NBUF):
                r, h = r0 + b // steps_per_row, b % steps_per_row
                gather(r, h, b).wait()
                store(r, h, b).start()
            for b in range(_NBUF):
                r, h = r0 + b // steps_per_row, b % steps_per_row
                store(r, h, b).wait()
                nr = r + rows_per_group

                @pl.when(nr < rows_per_w)
                def _():
                    gather(nr, h, b).start()

            return carry

        lax.fori_loop(0, rows_per_w // rows_per_group, loop_body, 0)

    return run


_NSPLIT = 2


def kernel(token_ids, weight):
    n_b, n_s = token_ids.shape
    dim = weight.shape[1]
    t = token_ids.astype(jnp.int32)
    run = _build(n_b // _NSPLIT, n_s, dim)
    step = n_b // _NSPLIT
    parts = [run(t[i * step : (i + 1) * step], weight) for i in range(_NSPLIT)]
    return jnp.concatenate(parts, axis=0)


# speedup vs baseline: 1.2434x; 1.1923x over previous
import functools

import jax
import jax.numpy as jnp
from jax import lax
from jax.experimental import pallas as pl
from jax.experimental.pallas import tpu as pltpu
from jax.experimental.pallas import tpu_sc as plsc

_NUM_CORES = 2
_NUM_SUBCORES = 16
_NW = _NUM_CORES * _NUM_SUBCORES
_CHUNK = 128
_NBUF = 4


@functools.lru_cache(maxsize=None)
def _build(n_rows, dim):
    rows_per_w = n_rows // _NW
    chunks_per_w = rows_per_w // _CHUNK
    n_groups = chunks_per_w // _NBUF
    mesh = plsc.VectorSubcoreMesh(core_axis_name="c", subcore_axis_name="s")

    @functools.partial(
        pl.kernel,
        mesh=mesh,
        out_type=jax.ShapeDtypeStruct((n_rows, dim), jnp.float32),
        scratch_types=(
            [pltpu.VMEM((chunks_per_w, _CHUNK), jnp.int32)]
            + [pltpu.VMEM((_CHUNK, dim), jnp.float32) for _ in range(_NBUF)]
            + [pltpu.SemaphoreType.DMA for _ in range(2 * _NBUF)]
        ),
        compiler_params=pltpu.CompilerParams(
            use_tc_tiling_on_sc=False, skip_device_barrier=True
        ),
    )
    def run(idx_hbm, table_hbm, out_hbm, idx_v, *bufs_and_sems):
        bufs = bufs_and_sems[:_NBUF]
        gsems = bufs_and_sems[_NBUF : 2 * _NBUF]
        osems = bufs_and_sems[2 * _NBUF :]
        wid = lax.axis_index("s") * _NUM_CORES + lax.axis_index("c")
        pltpu.sync_copy(idx_hbm.at[pl.ds(wid * chunks_per_w, chunks_per_w)], idx_v)
        base = wid * rows_per_w

        def gather(j, b):
            return pltpu.make_async_copy(table_hbm.at[idx_v.at[j]], bufs[b], gsems[b])

        def store(j, b):
            return pltpu.make_async_copy(
                bufs[b], out_hbm.at[pl.ds(base + j * _CHUNK, _CHUNK)], osems[b]
            )

        for b in range(_NBUF):
            gather(b, b).start()

        def loop_body(g, carry):
            j0 = g * _NBUF
            for b in range(_NBUF):
                gather(j0 + b, b).wait()
                store(j0 + b, b).start()
            for b in range(_NBUF):
                store(j0 + b, b).wait()
                nj = j0 + b + _NBUF

                @pl.when(nj < chunks_per_w)
                def _():
                    gather(nj, b).start()

            return carry

        lax.fori_loop(0, n_groups, loop_body, 0)

    return run


def kernel(token_ids, weight):
    n_rows = token_ids.size
    dim = weight.shape[1]
    idx = token_ids.reshape(n_rows // _CHUNK, _CHUNK).astype(jnp.int32)
    out = _build(n_rows, dim)(idx, weight)
    return out.reshape(token_ids.shape + (dim,))
